# Initial kernel scaffold; baseline (speedup 1.0000x reference)
import jax
import jax.numpy as jnp
from jax.experimental import pallas as pl


def kernel(x, edge_index, batch, W1, W2, Wm1, bm1, Wm2, bm2):
    raise NotImplementedError("write your pallas kernel here")


# speedup vs baseline: 13.1198x; 13.1198x over previous
import functools

import jax
import jax.numpy as jnp
from jax import lax
from jax.experimental import pallas as pl
from jax.experimental.pallas import tpu as pltpu
from jax.experimental.pallas import tpu_sc as plsc

N = 10000
E = 320000
D = 128
G = 64

NC = 2
NS = 16
NW = NC * NS
EPW = E // NW
K = 80
NCHUNK = EPW // K
RPT = 624
TAIL = N - NS * RPT
TAIL0 = NS * RPT
DRPT = 640
NPAD = NS * DRPT


@functools.cache
def _sc_kernels():
    mesh = plsc.VectorSubcoreMesh(core_axis_name="c", subcore_axis_name="s",
                                  num_cores=NC, num_subcores=NS)

    @functools.partial(
        pl.kernel,
        out_type=jax.ShapeDtypeStruct((NC, NPAD), jnp.float32),
        mesh=mesh,
        scratch_types=[
            pltpu.VMEM((1, K), jnp.int32),
            pltpu.VMEM((K,), jnp.float32),
            pltpu.VMEM((DRPT,), jnp.float32),
            pltpu.VMEM_SHARED((NPAD,), jnp.float32),
        ],
    )
    def sc_degree(dsti, out, didx, ones_v, deg_v, accum):
        c = lax.axis_index("c")
        s = lax.axis_index("s")
        wid = c * NS + s
        r0 = s * DRPT

        def fill(g, carry):
            deg_v[pl.ds(g * 16, 16)] = jnp.zeros((16,), jnp.float32)
            return carry

        lax.fori_loop(0, DRPT // 16, fill, 0)

        def fill1(g, carry):
            ones_v[pl.ds(g * 16, 16)] = jnp.ones((16,), jnp.float32)
            return carry

        lax.fori_loop(0, K // 16, fill1, 0)

        pltpu.sync_copy(deg_v.at[pl.ds(0, DRPT)], accum.at[pl.ds(r0, DRPT)])
        plsc.subcore_barrier()

        def body(j, carry):
            off = wid * EPW + j * K
            pltpu.sync_copy(dsti.at[pl.ds(off, K)], didx.at[0])
            pltpu.sync_copy(ones_v, accum.at[didx.at[0]], add=True)
            return carry

        lax.fori_loop(0, NCHUNK, body, 0)
        plsc.subcore_barrier()

        pltpu.sync_copy(accum.at[pl.ds(r0, DRPT)], deg_v.at[pl.ds(0, DRPT)])
        pltpu.sync_copy(deg_v.at[pl.ds(0, DRPT)], out.at[c].at[pl.ds(r0, DRPT)])

    @functools.partial(
        pl.kernel,
        out_type=jax.ShapeDtypeStruct((NC, N, D), jnp.float32),
        mesh=mesh,
        scratch_types=[
            pltpu.VMEM((K,), jnp.int32),
            pltpu.VMEM((1, K), jnp.int32),
            pltpu.VMEM((K, D), jnp.float32),
            pltpu.VMEM_SHARED((N, D), jnp.float32),
            pltpu.SemaphoreType.DMA,
        ],
    )
    def sc_scatter(hp, srci, dsti, zeros, out, sidx, didx, rows, accum, sem):
        c = lax.axis_index("c")
        s = lax.axis_index("s")
        wid = c * NS + s
        r0 = s * RPT
        pltpu.sync_copy(zeros.at[pl.ds(r0, RPT)], accum.at[pl.ds(r0, RPT)])

        @pl.when(s == NS - 1)
        def _():
            pltpu.sync_copy(zeros.at[pl.ds(TAIL0, TAIL)],
                            accum.at[pl.ds(TAIL0, TAIL)])

        plsc.subcore_barrier()

        def body(j, carry):
            off = wid * EPW + j * K
            pltpu.sync_copy(srci.at[pl.ds(off, K)], sidx)
            pltpu.async_copy(hp.at[sidx], rows, sem).wait()
            pltpu.sync_copy(dsti.at[pl.ds(off, K)], didx.at[0])
            pltpu.sync_copy(rows, accum.at[didx.at[0]], add=True)
            return carry

        lax.fori_loop(0, NCHUNK, body, 0)
        plsc.subcore_barrier()
        pltpu.sync_copy(accum.at[pl.ds(r0, RPT)], out.at[c].at[pl.ds(r0, RPT)])

        @pl.when(s == NS - 1)
        def _():
            pltpu.sync_copy(accum.at[pl.ds(TAIL0, TAIL)],
                            out.at[c].at[pl.ds(TAIL0, TAIL)])

    return sc_degree, sc_scatter


_BN = 1000
_NB = N // _BN


def _dinv_of(deg_ref):
    deg = deg_ref[0, :, 0:1] + deg_ref[1, :, 0:1] + 1.0
    return lax.rsqrt(deg)


def _tc_first(x, W1, degp):
    def body(x_ref, w_ref, deg_ref, out_ref):
        h = jnp.dot(x_ref[...], w_ref[...], preferred_element_type=jnp.float32)
        out_ref[...] = h * _dinv_of(deg_ref)

    return pl.pallas_call(
        body,
        grid=(_NB,),
        in_specs=[
            pl.BlockSpec((_BN, D), lambda i: (i, 0)),
            pl.BlockSpec((D, D), lambda i: (0, 0)),
            pl.BlockSpec((NC, _BN, 16), lambda i: (0, i, 0)),
        ],
        out_specs=pl.BlockSpec((_BN, D), lambda i: (i, 0)),
        out_shape=jax.ShapeDtypeStruct((N, D), jnp.float32),
    )(x, W1, degp)


def _tc_mid(a1p, h1p, degp, W2):
    def body(a_ref, h_ref, deg_ref, w_ref, out_ref):
        dinv = _dinv_of(deg_ref)
        t = jnp.maximum(dinv * (a_ref[0] + a_ref[1] + h_ref[...]), 0.0)
        out_ref[...] = jnp.dot(t, w_ref[...], preferred_element_type=jnp.float32) * dinv

    return pl.pallas_call(
        body,
        grid=(_NB,),
        in_specs=[
            pl.BlockSpec((NC, _BN, D), lambda i: (0, i, 0)),
            pl.BlockSpec((_BN, D), lambda i: (i, 0)),
            pl.BlockSpec((NC, _BN, 16), lambda i: (0, i, 0)),
            pl.BlockSpec((D, D), lambda i: (0, 0)),
        ],
        out_specs=pl.BlockSpec((_BN, D), lambda i: (i, 0)),
        out_shape=jax.ShapeDtypeStruct((N, D), jnp.float32),
    )(a1p, h1p, degp, W2)


def _tc_pool_mlp(a2p, h2p, degp, batch3, Wm1, bm1, Wm2, bm2):
    def body(a_ref, h_ref, deg_ref, b_ref, wm1_ref, bm1_ref, wm2_ref, bm2_ref,
             out_ref, sums_ref, cnts_ref):
        i = pl.program_id(0)

        @pl.when(i == 0)
        def _():
            sums_ref[...] = jnp.zeros_like(sums_ref)
            cnts_ref[...] = jnp.zeros_like(cnts_ref)

        dinv = _dinv_of(deg_ref)
        h2 = dinv * (a_ref[0] + a_ref[1] + h_ref[...])
        ids = b_ref[0, 0, :]
        gi = lax.broadcasted_iota(jnp.int32, (G, _BN), 0)
        onehot_t = (gi == ids[None, :]).astype(jnp.float32)
        sums_ref[...] += jnp.dot(onehot_t, h2, preferred_element_type=jnp.float32)
        cnts_ref[...] += jnp.sum(onehot_t, axis=1, keepdims=True)

        @pl.when(i == _NB - 1)
        def _():
            pooled = sums_ref[...] / jnp.maximum(cnts_ref[...], 1.0)
            z = jnp.dot(pooled, wm1_ref[...], preferred_element_type=jnp.float32)
            z = jnp.maximum(z + bm1_ref[...], 0.0)
            out_ref[...] = (jnp.dot(z, wm2_ref[...], preferred_element_type=jnp.float32)
                            + bm2_ref[...])

    return pl.pallas_call(
        body,
        grid=(_NB,),
        in_specs=[
            pl.BlockSpec((NC, _BN, D), lambda i: (0, i, 0)),
            pl.BlockSpec((_BN, D), lambda i: (i, 0)),
            pl.BlockSpec((NC, _BN, 16), lambda i: (0, i, 0)),
            pl.BlockSpec((1, 1, _BN), lambda i: (i, 0, 0)),
            pl.BlockSpec((D, D), lambda i: (0, 0)),
            pl.BlockSpec((1, D), lambda i: (0, 0)),
            pl.BlockSpec((D, D), lambda i: (0, 0)),
            pl.BlockSpec((1, D), lambda i: (0, 0)),
        ],
        out_specs=pl.BlockSpec((G, D), lambda i: (0, 0)),
        out_shape=jax.ShapeDtypeStruct((G, D), jnp.float32),
        scratch_shapes=[
            pltpu.VMEM((G, D), jnp.float32),
            pltpu.VMEM((G, 1), jnp.float32),
        ],
    )(a2p, h2p, degp, batch3, Wm1, bm1, Wm2, bm2)


def kernel(x, edge_index, batch, W1, W2, Wm1, bm1, Wm2, bm2):
    src = edge_index[0].astype(jnp.int32)
    dst = edge_index[1].astype(jnp.int32)

    zeros = jnp.zeros((N, D), jnp.float32)

    sc_degree, sc_scatter = _sc_kernels()
    deg1 = sc_degree(dst)[:, :N]
    degp = jnp.broadcast_to(deg1[:, :, None], (NC, N, 16))
    h1p = _tc_first(x, W1, degp)
    a1p = sc_scatter(h1p, src, dst, zeros)
    h2p = _tc_mid(a1p, h1p, degp, W2)
    a2p = sc_scatter(h2p, src, dst, zeros)
    return _tc_pool_mlp(a2p, h2p, degp, batch.reshape(_NB, 1, _BN),
                        Wm1, bm1.reshape(1, D), Wm2, bm2.reshape(1, D))


# speedup vs baseline: 22.5512x; 1.7189x over previous
import functools

import jax
import jax.numpy as jnp
from jax import lax
from jax.experimental import pallas as pl
from jax.experimental.pallas import tpu as pltpu
from jax.experimental.pallas import tpu_sc as plsc

N = 10000
E = 320000
D = 128
G = 64

NC = 2
NS = 16
NW = NC * NS
EPW = E // NW
K = 80
NCHUNK = EPW // K
K2 = 128
NCH2 = EPW // K2
TK = EPW - NCH2 * K2
RPT = 624
TAIL = N - NS * RPT
TAIL0 = NS * RPT
DRPT = 640
NPAD = NS * DRPT


@functools.cache
def _sc_kernels():
    mesh = plsc.VectorSubcoreMesh(core_axis_name="c", subcore_axis_name="s",
                                  num_cores=NC, num_subcores=NS)

    @functools.partial(
        pl.kernel,
        out_type=jax.ShapeDtypeStruct((NC, NPAD), jnp.float32),
        mesh=mesh,
        scratch_types=[
            pltpu.VMEM((1, K), jnp.int32),
            pltpu.VMEM((K,), jnp.float32),
            pltpu.VMEM((DRPT,), jnp.float32),
            pltpu.VMEM_SHARED((NPAD,), jnp.float32),
        ],
    )
    def sc_degree(dsti, out, didx, ones_v, deg_v, accum):
        c = lax.axis_index("c")
        s = lax.axis_index("s")
        wid = c * NS + s
        r0 = s * DRPT

        def fill(g, carry):
            deg_v[pl.ds(g * 16, 16)] = jnp.zeros((16,), jnp.float32)
            return carry

        lax.fori_loop(0, DRPT // 16, fill, 0)

        def fill1(g, carry):
            ones_v[pl.ds(g * 16, 16)] = jnp.ones((16,), jnp.float32)
            return carry

        lax.fori_loop(0, K // 16, fill1, 0)

        pltpu.sync_copy(deg_v.at[pl.ds(0, DRPT)], accum.at[pl.ds(r0, DRPT)])
        plsc.subcore_barrier()

        def body(j, carry):
            off = wid * EPW + j * K
            pltpu.sync_copy(dsti.at[pl.ds(off, K)], didx.at[0])
            pltpu.sync_copy(ones_v, accum.at[didx.at[0]], add=True)
            return carry

        lax.fori_loop(0, NCHUNK, body, 0)
        plsc.subcore_barrier()

        pltpu.sync_copy(accum.at[pl.ds(r0, DRPT)], deg_v.at[pl.ds(0, DRPT)])
        pltpu.sync_copy(deg_v.at[pl.ds(0, DRPT)], out.at[c].at[pl.ds(r0, DRPT)])

    @functools.partial(
        pl.kernel,
        out_type=jax.ShapeDtypeStruct((NC, N, D), jnp.float32),
        mesh=mesh,
        scratch_types=[
            pltpu.VMEM((2, K2), jnp.int32),
            pltpu.VMEM((2, K2), jnp.int32),
            pltpu.VMEM((2, K2, D), jnp.float32),
            pltpu.VMEM((TK,), jnp.int32),
            pltpu.VMEM((1, TK), jnp.int32),
            pltpu.VMEM((TK, D), jnp.float32),
            pltpu.VMEM_SHARED((N, D), jnp.float32),
            pltpu.SemaphoreType.DMA,
            pltpu.SemaphoreType.DMA,
        ],
    )
    def sc_scatter(hp, srci, dsti, zeros, out,
                   sidx2, didx2, rows2, sidx_t, didx_t, rows_t, accum,
                   sem0, sem1):
        c = lax.axis_index("c")
        s = lax.axis_index("s")
        wid = c * NS + s
        r0 = s * RPT
        ebase = wid * EPW
        sems = (sem0, sem1)

        pltpu.sync_copy(zeros.at[pl.ds(r0, RPT)], accum.at[pl.ds(r0, RPT)])

        @pl.when(s == NS - 1)
        def _():
            pltpu.sync_copy(zeros.at[pl.ds(TAIL0, TAIL)],
                            accum.at[pl.ds(TAIL0, TAIL)])

        plsc.subcore_barrier()

        def g_start(j, b):
            off = ebase + j * K2
            pltpu.sync_copy(srci.at[pl.ds(off, K2)], sidx2.at[b])
            pltpu.sync_copy(dsti.at[pl.ds(off, K2)], didx2.at[b])
            pltpu.async_copy(hp.at[sidx2.at[b]], rows2.at[b], sems[b])

        def g_wait_scat(b):
            pltpu.make_async_copy(hp.at[sidx2.at[b]], rows2.at[b],
                                  sems[b]).wait()
            pltpu.sync_copy(rows2.at[b], accum.at[didx2.at[b]], add=True)

        g_start(0, 0)

        def body(p, carry):
            g_start(2 * p + 1, 1)
            g_wait_scat(0)

            @pl.when(p < NCH2 // 2 - 1)
            def _():
                g_start(2 * p + 2, 0)

            g_wait_scat(1)
            return carry

        lax.fori_loop(0, NCH2 // 2, body, 0)

        toff = ebase + NCH2 * K2
        pltpu.sync_copy(srci.at[pl.ds(toff, TK)], sidx_t)
        pltpu.async_copy(hp.at[sidx_t], rows_t, sem0).wait()
        pltpu.sync_copy(dsti.at[pl.ds(toff, TK)], didx_t.at[0])
        pltpu.sync_copy(rows_t, accum.at[didx_t.at[0]], add=True)

        plsc.subcore_barrier()
        pltpu.sync_copy(accum.at[pl.ds(r0, RPT)], out.at[c].at[pl.ds(r0, RPT)])

        @pl.when(s == NS - 1)
        def _():
            pltpu.sync_copy(accum.at[pl.ds(TAIL0, TAIL)],
                            out.at[c].at[pl.ds(TAIL0, TAIL)])

    return sc_degree, sc_scatter


_BN = 1000
_NB = N // _BN


def _dinv_of(deg_ref):
    deg = deg_ref[0, :, 0:1] + deg_ref[1, :, 0:1] + 1.0
    return lax.rsqrt(deg)


def _tc_first(x, W1, degp):
    def body(x_ref, w_ref, deg_ref, out_ref):
        h = jnp.dot(x_ref[...], w_ref[...], preferred_element_type=jnp.float32)
        out_ref[...] = h * _dinv_of(deg_ref)

    return pl.pallas_call(
        body,
        grid=(_NB,),
        in_specs=[
            pl.BlockSpec((_BN, D), lambda i: (i, 0)),
            pl.BlockSpec((D, D), lambda i: (0, 0)),
            pl.BlockSpec((NC, _BN, 16), lambda i: (0, i, 0)),
        ],
        out_specs=pl.BlockSpec((_BN, D), lambda i: (i, 0)),
        out_shape=jax.ShapeDtypeStruct((N, D), jnp.float32),
    )(x, W1, degp)


def _tc_mid(a1p, h1p, degp, W2):
    def body(a_ref, h_ref, deg_ref, w_ref, out_ref):
        dinv = _dinv_of(deg_ref)
        t = jnp.maximum(dinv * (a_ref[0] + a_ref[1] + h_ref[...]), 0.0)
        out_ref[...] = jnp.dot(t, w_ref[...], preferred_element_type=jnp.float32) * dinv

    return pl.pallas_call(
        body,
        grid=(_NB,),
        in_specs=[
            pl.BlockSpec((NC, _BN, D), lambda i: (0, i, 0)),
            pl.BlockSpec((_BN, D), lambda i: (i, 0)),
            pl.BlockSpec((NC, _BN, 16), lambda i: (0, i, 0)),
            pl.BlockSpec((D, D), lambda i: (0, 0)),
        ],
        out_specs=pl.BlockSpec((_BN, D), lambda i: (i, 0)),
        out_shape=jax.ShapeDtypeStruct((N, D), jnp.float32),
    )(a1p, h1p, degp, W2)


def _tc_pool_mlp(a2p, h2p, degp, batch3, Wm1, bm1, Wm2, bm2):
    def body(a_ref, h_ref, deg_ref, b_ref, wm1_ref, bm1_ref, wm2_ref, bm2_ref,
             out_ref, sums_ref, cnts_ref):
        i = pl.program_id(0)

        @pl.when(i == 0)
        def _():
            sums_ref[...] = jnp.zeros_like(sums_ref)
            cnts_ref[...] = jnp.zeros_like(cnts_ref)

        dinv = _dinv_of(deg_ref)
        h2 = dinv * (a_ref[0] + a_ref[1] + h_ref[...])
        ids = b_ref[0, 0, :]
        gi = lax.broadcasted_iota(jnp.int32, (G, _BN), 0)
        onehot_t = (gi == ids[None, :]).astype(jnp.float32)
        sums_ref[...] += jnp.dot(onehot_t, h2, preferred_element_type=jnp.float32)
        cnts_ref[...] += jnp.sum(onehot_t, axis=1, keepdims=True)

        @pl.when(i == _NB - 1)
        def _():
            pooled = sums_ref[...] / jnp.maximum(cnts_ref[...], 1.0)
            z = jnp.dot(pooled, wm1_ref[...], preferred_element_type=jnp.float32)
            z = jnp.maximum(z + bm1_ref[...], 0.0)
            out_ref[...] = (jnp.dot(z, wm2_ref[...], preferred_element_type=jnp.float32)
                            + bm2_ref[...])

    return pl.pallas_call(
        body,
        grid=(_NB,),
        in_specs=[
            pl.BlockSpec((NC, _BN, D), lambda i: (0, i, 0)),
            pl.BlockSpec((_BN, D), lambda i: (i, 0)),
            pl.BlockSpec((NC, _BN, 16), lambda i: (0, i, 0)),
            pl.BlockSpec((1, 1, _BN), lambda i: (i, 0, 0)),
            pl.BlockSpec((D, D), lambda i: (0, 0)),
            pl.BlockSpec((1, D), lambda i: (0, 0)),
            pl.BlockSpec((D, D), lambda i: (0, 0)),
            pl.BlockSpec((1, D), lambda i: (0, 0)),
        ],
        out_specs=pl.BlockSpec((G, D), lambda i: (0, 0)),
        out_shape=jax.ShapeDtypeStruct((G, D), jnp.float32),
        scratch_shapes=[
            pltpu.VMEM((G, D), jnp.float32),
            pltpu.VMEM((G, 1), jnp.float32),
        ],
    )(a2p, h2p, degp, batch3, Wm1, bm1, Wm2, bm2)


def kernel(x, edge_index, batch, W1, W2, Wm1, bm1, Wm2, bm2):
    src = edge_index[0].astype(jnp.int32)
    dst = edge_index[1].astype(jnp.int32)

    zeros = jnp.zeros((N, D), jnp.float32)

    sc_degree, sc_scatter = _sc_kernels()
    deg1 = sc_degree(dst)[:, :N]
    degp = jnp.broadcast_to(deg1[:, :, None], (NC, N, 16))
    h1p = _tc_first(x, W1, degp)
    a1p = sc_scatter(h1p, src, dst, zeros)
    h2p = _tc_mid(a1p, h1p, degp, W2)
    a2p = sc_scatter(h2p, src, dst, zeros)
    return _tc_pool_mlp(a2p, h2p, degp, batch.reshape(_NB, 1, _BN),
                        Wm1, bm1.reshape(1, D), Wm2, bm2.reshape(1, D))


# speedup vs baseline: 24.8389x; 1.1014x over previous
import functools

import jax
import jax.numpy as jnp
from jax import lax
from jax.experimental import pallas as pl
from jax.experimental.pallas import tpu as pltpu
from jax.experimental.pallas import tpu_sc as plsc

N = 10000
E = 320000
D = 128
G = 64

NC = 2
NS = 16
NW = NC * NS
EPW = E // NW
K = 80
NCHUNK = EPW // K
K2 = 128
NCH2 = EPW // K2
TK = EPW - NCH2 * K2
RPT = 624
TAIL = N - NS * RPT
TAIL0 = NS * RPT
DRPT = 640
NPAD = NS * DRPT


@functools.cache
def _sc_kernels():
    mesh = plsc.VectorSubcoreMesh(core_axis_name="c", subcore_axis_name="s",
                                  num_cores=NC, num_subcores=NS)

    @functools.partial(
        pl.kernel,
        out_type=jax.ShapeDtypeStruct((NC, NPAD), jnp.float32),
        mesh=mesh,
        scratch_types=[
            pltpu.VMEM((2, K2), jnp.int32),
            pltpu.VMEM((1, TK), jnp.int32),
            pltpu.VMEM((K2,), jnp.float32),
            pltpu.VMEM((DRPT,), jnp.float32),
            pltpu.VMEM_SHARED((NPAD,), jnp.float32),
            pltpu.SemaphoreType.DMA,
            pltpu.SemaphoreType.DMA,
        ],
    )
    def sc_degree(dsti, out, didx2, didx_t, ones_v, deg_v, accum, sem0, sem1):
        c = lax.axis_index("c")
        s = lax.axis_index("s")
        wid = c * NS + s
        r0 = s * DRPT
        ebase = wid * EPW
        sems = (sem0, sem1)

        def fill(g, carry):
            deg_v[pl.ds(g * 16, 16)] = jnp.zeros((16,), jnp.float32)
            return carry

        lax.fori_loop(0, DRPT // 16, fill, 0)

        def fill1(g, carry):
            ones_v[pl.ds(g * 16, 16)] = jnp.ones((16,), jnp.float32)
            return carry

        lax.fori_loop(0, K2 // 16, fill1, 0)

        pltpu.sync_copy(deg_v.at[pl.ds(0, DRPT)], accum.at[pl.ds(r0, DRPT)])
        plsc.subcore_barrier()

        def i_start(j, b):
            pltpu.async_copy(dsti.at[pl.ds(ebase + j * K2, K2)],
                             didx2.at[b], sems[b])

        def i_wait_scat(j, b):
            pltpu.make_async_copy(dsti.at[pl.ds(ebase + j * K2, K2)],
                                  didx2.at[b], sems[b]).wait()
            pltpu.sync_copy(ones_v, accum.at[didx2.at[b]], add=True)

        i_start(0, 0)

        def body(p, carry):
            i_start(2 * p + 1, 1)
            i_wait_scat(2 * p, 0)

            @pl.when(p < NCH2 // 2 - 1)
            def _():
                i_start(2 * p + 2, 0)

            i_wait_scat(2 * p + 1, 1)
            return carry

        lax.fori_loop(0, NCH2 // 2, body, 0)

        toff = ebase + NCH2 * K2
        pltpu.sync_copy(dsti.at[pl.ds(toff, TK)], didx_t.at[0])
        pltpu.sync_copy(ones_v.at[pl.ds(0, TK)],
                        accum.at[didx_t.at[0]], add=True)

        plsc.subcore_barrier()
        pltpu.sync_copy(accum.at[pl.ds(r0, DRPT)], deg_v.at[pl.ds(0, DRPT)])
        pltpu.sync_copy(deg_v.at[pl.ds(0, DRPT)], out.at[c].at[pl.ds(r0, DRPT)])

    @functools.partial(
        pl.kernel,
        out_type=jax.ShapeDtypeStruct((NC, N, D), jnp.float32),
        mesh=mesh,
        scratch_types=[
            pltpu.VMEM((2, K2), jnp.int32),
            pltpu.VMEM((2, K2), jnp.int32),
            pltpu.VMEM((2, K2, D), jnp.float32),
            pltpu.VMEM((TK,), jnp.int32),
            pltpu.VMEM((1, TK), jnp.int32),
            pltpu.VMEM((TK, D), jnp.float32),
            pltpu.VMEM_SHARED((N, D), jnp.float32),
            pltpu.SemaphoreType.DMA,
            pltpu.SemaphoreType.DMA,
        ],
    )
    def sc_scatter(hp, srci, dsti, zeros, out,
                   sidx2, didx2, rows2, sidx_t, didx_t, rows_t, accum,
                   sem0, sem1):
        c = lax.axis_index("c")
        s = lax.axis_index("s")
        wid = c * NS + s
        r0 = s * RPT
        ebase = wid * EPW
        sems = (sem0, sem1)

        pltpu.sync_copy(zeros.at[pl.ds(r0, RPT)], accum.at[pl.ds(r0, RPT)])

        @pl.when(s == NS - 1)
        def _():
            pltpu.sync_copy(zeros.at[pl.ds(TAIL0, TAIL)],
                            accum.at[pl.ds(TAIL0, TAIL)])

        plsc.subcore_barrier()

        def g_start(j, b):
            off = ebase + j * K2
            pltpu.sync_copy(srci.at[pl.ds(off, K2)], sidx2.at[b])
            pltpu.sync_copy(dsti.at[pl.ds(off, K2)], didx2.at[b])
            pltpu.async_copy(hp.at[sidx2.at[b]], rows2.at[b], sems[b])

        def g_wait_scat(b):
            pltpu.make_async_copy(hp.at[sidx2.at[b]], rows2.at[b],
                                  sems[b]).wait()
            pltpu.sync_copy(rows2.at[b], accum.at[didx2.at[b]], add=True)

        g_start(0, 0)

        def body(p, carry):
            g_start(2 * p + 1, 1)
            g_wait_scat(0)

            @pl.when(p < NCH2 // 2 - 1)
            def _():
                g_start(2 * p + 2, 0)

            g_wait_scat(1)
            return carry

        lax.fori_loop(0, NCH2 // 2, body, 0)

        toff = ebase + NCH2 * K2
        pltpu.sync_copy(srci.at[pl.ds(toff, TK)], sidx_t)
        pltpu.async_copy(hp.at[sidx_t], rows_t, sem0).wait()
        pltpu.sync_copy(dsti.at[pl.ds(toff, TK)], didx_t.at[0])
        pltpu.sync_copy(rows_t, accum.at[didx_t.at[0]], add=True)

        plsc.subcore_barrier()
        pltpu.sync_copy(accum.at[pl.ds(r0, RPT)], out.at[c].at[pl.ds(r0, RPT)])

        @pl.when(s == NS - 1)
        def _():
            pltpu.sync_copy(accum.at[pl.ds(TAIL0, TAIL)],
                            out.at[c].at[pl.ds(TAIL0, TAIL)])

    return sc_degree, sc_scatter


_BN = 1000
_NB = N // _BN


def _dinv_of(deg_ref):
    deg = deg_ref[0, :, 0:1] + deg_ref[1, :, 0:1] + 1.0
    return lax.rsqrt(deg)


def _tc_first(x, W1, degp):
    def body(x_ref, w_ref, deg_ref, out_ref):
        h = jnp.dot(x_ref[...], w_ref[...], preferred_element_type=jnp.float32)
        out_ref[...] = h * _dinv_of(deg_ref)

    return pl.pallas_call(
        body,
        grid=(_NB,),
        in_specs=[
            pl.BlockSpec((_BN, D), lambda i: (i, 0)),
            pl.BlockSpec((D, D), lambda i: (0, 0)),
            pl.BlockSpec((NC, _BN, 16), lambda i: (0, i, 0)),
        ],
        out_specs=pl.BlockSpec((_BN, D), lambda i: (i, 0)),
        out_shape=jax.ShapeDtypeStruct((N, D), jnp.float32),
    )(x, W1, degp)


def _tc_mid(a1p, h1p, degp, W2):
    def body(a_ref, h_ref, deg_ref, w_ref, out_ref):
        dinv = _dinv_of(deg_ref)
        t = jnp.maximum(dinv * (a_ref[0] + a_ref[1] + h_ref[...]), 0.0)
        out_ref[...] = jnp.dot(t, w_ref[...], preferred_element_type=jnp.float32) * dinv

    return pl.pallas_call(
        body,
        grid=(_NB,),
        in_specs=[
            pl.BlockSpec((NC, _BN, D), lambda i: (0, i, 0)),
            pl.BlockSpec((_BN, D), lambda i: (i, 0)),
            pl.BlockSpec((NC, _BN, 16), lambda i: (0, i, 0)),
            pl.BlockSpec((D, D), lambda i: (0, 0)),
        ],
        out_specs=pl.BlockSpec((_BN, D), lambda i: (i, 0)),
        out_shape=jax.ShapeDtypeStruct((N, D), jnp.float32),
    )(a1p, h1p, degp, W2)


def _tc_pool_mlp(a2p, h2p, degp, batch3, Wm1, bm1, Wm2, bm2):
    def body(a_ref, h_ref, deg_ref, b_ref, wm1_ref, bm1_ref, wm2_ref, bm2_ref,
             out_ref, sums_ref, cnts_ref):
        i = pl.program_id(0)

        @pl.when(i == 0)
        def _():
            sums_ref[...] = jnp.zeros_like(sums_ref)
            cnts_ref[...] = jnp.zeros_like(cnts_ref)

        dinv = _dinv_of(deg_ref)
        h2 = dinv * (a_ref[0] + a_ref[1] + h_ref[...])
        ids = b_ref[0, 0, :]
        gi = lax.broadcasted_iota(jnp.int32, (G, _BN), 0)
        onehot_t = (gi == ids[None, :]).astype(jnp.float32)
        sums_ref[...] += jnp.dot(onehot_t, h2, preferred_element_type=jnp.float32)
        cnts_ref[...] += jnp.sum(onehot_t, axis=1, keepdims=True)

        @pl.when(i == _NB - 1)
        def _():
            pooled = sums_ref[...] / jnp.maximum(cnts_ref[...], 1.0)
            z = jnp.dot(pooled, wm1_ref[...], preferred_element_type=jnp.float32)
            z = jnp.maximum(z + bm1_ref[...], 0.0)
            out_ref[...] = (jnp.dot(z, wm2_ref[...], preferred_element_type=jnp.float32)
                            + bm2_ref[...])

    return pl.pallas_call(
        body,
        grid=(_NB,),
        in_specs=[
            pl.BlockSpec((NC, _BN, D), lambda i: (0, i, 0)),
            pl.BlockSpec((_BN, D), lambda i: (i, 0)),
            pl.BlockSpec((NC, _BN, 16), lambda i: (0, i, 0)),
            pl.BlockSpec((1, 1, _BN), lambda i: (i, 0, 0)),
            pl.BlockSpec((D, D), lambda i: (0, 0)),
            pl.BlockSpec((1, D), lambda i: (0, 0)),
            pl.BlockSpec((D, D), lambda i: (0, 0)),
            pl.BlockSpec((1, D), lambda i: (0, 0)),
        ],
        out_specs=pl.BlockSpec((G, D), lambda i: (0, 0)),
        out_shape=jax.ShapeDtypeStruct((G, D), jnp.float32),
        scratch_shapes=[
            pltpu.VMEM((G, D), jnp.float32),
            pltpu.VMEM((G, 1), jnp.float32),
        ],
    )(a2p, h2p, degp, batch3, Wm1, bm1, Wm2, bm2)


def kernel(x, edge_index, batch, W1, W2, Wm1, bm1, Wm2, bm2):
    src = edge_index[0].astype(jnp.int32)
    dst = edge_index[1].astype(jnp.int32)

    zeros = jnp.zeros((N, D), jnp.float32)

    sc_degree, sc_scatter = _sc_kernels()
    deg1 = sc_degree(dst)[:, :N]
    degp = jnp.broadcast_to(deg1[:, :, None], (NC, N, 16))
    h1p = _tc_first(x, W1, degp)
    a1p = sc_scatter(h1p, src, dst, zeros)
    h2p = _tc_mid(a1p, h1p, degp, W2)
    a2p = sc_scatter(h2p, src, dst, zeros)
    return _tc_pool_mlp(a2p, h2p, degp, batch.reshape(_NB, 1, _BN),
                        Wm1, bm1.reshape(1, D), Wm2, bm2.reshape(1, D))
